# baseline (speedup 1.0000x reference)
import functools

import jax
import jax.numpy as jnp
from jax import lax
from jax.experimental import pallas as pl
from jax.experimental.pallas import tpu as pltpu
from jax.experimental.pallas import tpu_sc as plsc

N = 10000
E = 320000
D = 128

NC = 2
NS = 16
NW = NC * NS
CHUNK = 128
NCHUNK = 80
NPAIR = NCHUNK // 2
EPW = CHUNK * NCHUNK
E_PAD = EPW * NW
ACC_ROWS = 10240
ROWS_PER_TILE = ACC_ROWS // NS


def _leaky(h):
    return jnp.where(h >= 0, h, 0.01 * h)


def _deg_body(dst_hbm, ones_hbm, zero_hbm, out_hbm, idx_v, ones_v, acc_sh, sem):
    c = lax.axis_index("c")
    s = lax.axis_index("s")
    wid = c * NS + s
    pltpu.sync_copy(zero_hbm, acc_sh.at[pl.ds(s * ROWS_PER_TILE, ROWS_PER_TILE)])
    pltpu.sync_copy(ones_hbm, ones_v)
    pltpu.sync_copy(dst_hbm.at[wid], idx_v)
    plsc.subcore_barrier()

    def body(t, _):
        pltpu.async_copy(ones_v, acc_sh.at[idx_v.at[t]], sem, add=True)
        return 0

    lax.fori_loop(0, NCHUNK, body, 0)

    def drain(t, _):
        pltpu.make_async_copy(ones_v, acc_sh.at[idx_v.at[0]], sem).wait()
        return 0

    lax.fori_loop(0, NCHUNK, drain, 0)
    plsc.subcore_barrier()
    pltpu.sync_copy(
        acc_sh.at[pl.ds(s * ROWS_PER_TILE, ROWS_PER_TILE)],
        out_hbm.at[c, pl.ds(s * ROWS_PER_TILE, ROWS_PER_TILE)],
    )


@functools.cache
def _deg_kernel():
    mesh = plsc.VectorSubcoreMesh(
        core_axis_name="c", subcore_axis_name="s",
        num_cores=NC, num_subcores=NS,
    )
    return pl.kernel(
        _deg_body,
        out_type=jax.ShapeDtypeStruct((NC, ACC_ROWS, D), jnp.float32),
        mesh=mesh,
        scratch_types=[
            pltpu.VMEM((NCHUNK, CHUNK), jnp.int32),
            pltpu.VMEM((CHUNK, D), jnp.float32),
            pltpu.VMEM_SHARED((ACC_ROWS, D), jnp.float32),
            pltpu.SemaphoreType.DMA,
        ],
    )


def _agg_body(g_hbm, src_hbm, dst_hbm, zrows_hbm, out_hbm,
              si_v, di_v, rows0, rows1, acc_sh, sg0, sg1, ss0, ss1, s_si):
    c = lax.axis_index("c")
    s = lax.axis_index("s")
    wid = c * NS + s
    pltpu.sync_copy(zrows_hbm, acc_sh.at[pl.ds(s * ROWS_PER_TILE, ROWS_PER_TILE)])
    pltpu.sync_copy(dst_hbm.at[wid], di_v)
    plsc.subcore_barrier()

    pltpu.sync_copy(src_hbm.at[wid, pl.ds(0, 2)], si_v.at[0])
    pltpu.async_copy(src_hbm.at[wid, pl.ds(2, 2)], si_v.at[1], s_si)
    pltpu.async_copy(g_hbm.at[si_v.at[0, 0]], rows0, sg0)
    pltpu.async_copy(g_hbm.at[si_v.at[0, 1]], rows1, sg1)

    def body(i, _):
        t0 = 2 * i
        pltpu.make_async_copy(g_hbm.at[si_v.at[0, 0]], rows0, sg0).wait()
        pltpu.async_copy(rows0, acc_sh.at[di_v.at[t0]], ss0, add=True)
        pltpu.make_async_copy(g_hbm.at[si_v.at[0, 1]], rows1, sg1).wait()
        pltpu.async_copy(rows1, acc_sh.at[di_v.at[t0 + 1]], ss1, add=True)

        @pl.when(i + 1 < NPAIR)
        def _prefetch():
            slot = lax.rem(i + 1, 2)
            pltpu.make_async_copy(
                src_hbm.at[wid, pl.ds(0, 2)], si_v.at[0], s_si).wait()
            pltpu.make_async_copy(rows0, acc_sh.at[di_v.at[t0]], ss0).wait()
            pltpu.async_copy(g_hbm.at[si_v.at[slot, 0]], rows0, sg0)
            pltpu.make_async_copy(rows1, acc_sh.at[di_v.at[t0 + 1]], ss1).wait()
            pltpu.async_copy(g_hbm.at[si_v.at[slot, 1]], rows1, sg1)

            @pl.when(i + 2 < NPAIR)
            def _fetch_next():
                pltpu.async_copy(
                    src_hbm.at[wid, pl.ds(2 * i + 4, 2)],
                    si_v.at[lax.rem(i, 2)], s_si)

        return 0

    lax.fori_loop(0, NPAIR, body, 0)
    pltpu.make_async_copy(rows0, acc_sh.at[di_v.at[0]], ss0).wait()
    pltpu.make_async_copy(rows1, acc_sh.at[di_v.at[1]], ss1).wait()
    plsc.subcore_barrier()
    pltpu.sync_copy(
        acc_sh.at[pl.ds(s * ROWS_PER_TILE, ROWS_PER_TILE)],
        out_hbm.at[c, pl.ds(s * ROWS_PER_TILE, ROWS_PER_TILE)],
    )


@functools.cache
def _agg_kernel():
    mesh = plsc.VectorSubcoreMesh(
        core_axis_name="c", subcore_axis_name="s",
        num_cores=NC, num_subcores=NS,
    )
    return pl.kernel(
        _agg_body,
        out_type=jax.ShapeDtypeStruct((NC, ACC_ROWS, D), jnp.float32),
        mesh=mesh,
        scratch_types=[
            pltpu.VMEM((2, 2, CHUNK), jnp.int32),
            pltpu.VMEM((NCHUNK, CHUNK), jnp.int32),
            pltpu.VMEM((CHUNK, D), jnp.float32),
            pltpu.VMEM((CHUNK, D), jnp.float32),
            pltpu.VMEM_SHARED((ACC_ROWS, D), jnp.float32),
            pltpu.SemaphoreType.DMA,
            pltpu.SemaphoreType.DMA,
            pltpu.SemaphoreType.DMA,
            pltpu.SemaphoreType.DMA,
            pltpu.SemaphoreType.DMA,
        ],
    )


def _tc1_body(x_ref, w1_ref, b1_ref, w2_ref, b2_ref, h_ref):
    h = _leaky(jnp.dot(x_ref[...], w1_ref[...],
                       preferred_element_type=jnp.float32) + b1_ref[...])
    h_ref[...] = _leaky(jnp.dot(h, w2_ref[...],
                                preferred_element_type=jnp.float32) + b2_ref[...])


def _tc2_body(degp_ref, h_ref, wc_ref, g_ref, dinv_ref):
    deg = degp_ref[0, :N, 0:1] + degp_ref[1, :N, 0:1] + 1.0
    dinv = lax.rsqrt(deg)
    dinv_ref[...] = dinv
    g_ref[...] = dinv * jnp.dot(h_ref[...], wc_ref[...],
                                preferred_element_type=jnp.float32)


def _tc3_body(s_ref, g_ref, dinv_ref, bc_ref, wc_ref, emb_ref, g2_ref):
    ssum = s_ref[0, :N, :] + s_ref[1, :N, :] + g_ref[...]
    dinv = dinv_ref[...]
    emb = _leaky(dinv * ssum + bc_ref[...])
    emb_ref[...] = emb
    g2_ref[...] = dinv * jnp.dot(emb, wc_ref[...],
                                 preferred_element_type=jnp.float32)


def _tc4_body(s_ref, g_ref, dinv_ref, bc_ref, wp_ref, bp_ref, emb_ref, o_ref):
    ssum = s_ref[0, :N, :] + s_ref[1, :N, :] + g_ref[...]
    emb = _leaky(dinv_ref[...] * ssum + bc_ref[...])
    emb_ref[...] = emb
    o_ref[...] = jnp.dot(emb, wp_ref[...],
                         preferred_element_type=jnp.float32) + bp_ref[...]


_f32 = jnp.float32


def _tc1(x, W1, b1, W2, b2):
    return pl.pallas_call(
        _tc1_body, out_shape=jax.ShapeDtypeStruct((N, D), _f32)
    )(x, W1, b1, W2, b2)


def _tc2(degp, h, Wc):
    return pl.pallas_call(
        _tc2_body,
        out_shape=(jax.ShapeDtypeStruct((N, D), _f32),
                   jax.ShapeDtypeStruct((N, 1), _f32)),
    )(degp, h, Wc)


def _tc3(S, g, dinv, bc, Wc):
    return pl.pallas_call(
        _tc3_body,
        out_shape=(jax.ShapeDtypeStruct((N, D), _f32),
                   jax.ShapeDtypeStruct((N, D), _f32)),
    )(S, g, dinv, bc, Wc)


def _tc4(S, g, dinv, bc, Wp, bp):
    return pl.pallas_call(
        _tc4_body,
        out_shape=(jax.ShapeDtypeStruct((N, D), _f32),
                   jax.ShapeDtypeStruct((N, 1), _f32)),
    )(S, g, dinv, bc, Wp, bp)


@jax.jit
def kernel(x, edge_index, W1, b1, W2, b2, Wc1, bc1, Wc2, bc2, Wp, bp):
    pad = E_PAD - E
    src = jnp.concatenate([edge_index[0], jnp.zeros((pad,), jnp.int32)])
    dst = jnp.concatenate([edge_index[1], jnp.full((pad,), N, jnp.int32)])
    src = src.reshape(NW, NCHUNK, CHUNK)
    dst = dst.reshape(NW, NCHUNK, CHUNK)
    ones128 = jnp.ones((CHUNK, D), _f32)
    zrows = jnp.zeros((ROWS_PER_TILE, D), _f32)

    h = _tc1(x, W1, b1.reshape(1, D), W2, b2.reshape(1, D))
    degp = _deg_kernel()(dst, ones128, zrows)
    g1, dinv = _tc2(degp, h, Wc1)
    S1 = _agg_kernel()(g1, src, dst, zrows)
    emb1, g2 = _tc3(S1, g1, dinv, bc1.reshape(1, D), Wc2)
    S2 = _agg_kernel()(g2, src, dst, zrows)
    emb2, o = _tc4(S2, g2, dinv, bc2.reshape(1, D), Wp, bp.reshape(1, 1))
    return (o.reshape(N), emb1, emb2)


# speedup vs baseline: 2.4457x; 2.4457x over previous
import functools

import jax
import jax.numpy as jnp
from jax import lax
from jax.experimental import pallas as pl
from jax.experimental.pallas import tpu as pltpu
from jax.experimental.pallas import tpu_sc as plsc

N = 10000
E = 320000
D = 128

NC = 2
NS = 16
NW = NC * NS
CHUNK = 128
NCHUNK = 80
NPAIR = NCHUNK // 2
EPW = CHUNK * NCHUNK
E_PAD = EPW * NW
ACC_ROWS = 10240
ROWS_PER_TILE = ACC_ROWS // NS


def _leaky(h):
    return jnp.where(h >= 0, h, 0.01 * h)


def _deg_body(dst_hbm, ones_hbm, zero_hbm, out_hbm, idx_v, ones_v, acc_sh, sem):
    c = lax.axis_index("c")
    s = lax.axis_index("s")
    wid = c * NS + s
    pltpu.sync_copy(zero_hbm, acc_sh.at[pl.ds(s * ROWS_PER_TILE, ROWS_PER_TILE)])
    pltpu.sync_copy(ones_hbm, ones_v)
    pltpu.sync_copy(dst_hbm.at[wid], idx_v)
    plsc.subcore_barrier()

    def body(t, _):
        pltpu.async_copy(ones_v, acc_sh.at[idx_v.at[t]], sem, add=True)
        return 0

    lax.fori_loop(0, NCHUNK, body, 0)

    def drain(t, _):
        pltpu.make_async_copy(ones_v, acc_sh.at[idx_v.at[0]], sem).wait()
        return 0

    lax.fori_loop(0, NCHUNK, drain, 0)
    plsc.subcore_barrier()
    pltpu.sync_copy(
        acc_sh.at[pl.ds(s * ROWS_PER_TILE, ROWS_PER_TILE)],
        out_hbm.at[c, pl.ds(s * ROWS_PER_TILE, ROWS_PER_TILE)],
    )


@functools.cache
def _deg_kernel():
    mesh = plsc.VectorSubcoreMesh(
        core_axis_name="c", subcore_axis_name="s",
        num_cores=NC, num_subcores=NS,
    )
    return pl.kernel(
        _deg_body,
        out_type=jax.ShapeDtypeStruct((NC, ACC_ROWS, D), jnp.float32),
        mesh=mesh,
        scratch_types=[
            pltpu.VMEM((NCHUNK, CHUNK), jnp.int32),
            pltpu.VMEM((CHUNK, D), jnp.float32),
            pltpu.VMEM_SHARED((ACC_ROWS, D), jnp.float32),
            pltpu.SemaphoreType.DMA,
        ],
    )


def _agg_body(g_hbm, src_hbm, dst_hbm, zrows_hbm, out_hbm,
              si_v, di_v, rows0, rows1, acc_sh, sg0, sg1, ss0, ss1, s_si):
    c = lax.axis_index("c")
    s = lax.axis_index("s")
    wid = c * NS + s
    pltpu.sync_copy(zrows_hbm, acc_sh.at[pl.ds(s * ROWS_PER_TILE, ROWS_PER_TILE)])
    pltpu.sync_copy(dst_hbm.at[wid], di_v)
    plsc.subcore_barrier()

    pltpu.sync_copy(src_hbm.at[wid, pl.ds(0, 2)], si_v.at[0])
    pltpu.async_copy(src_hbm.at[wid, pl.ds(2, 2)], si_v.at[1], s_si)
    pltpu.async_copy(g_hbm.at[si_v.at[0, 0]], rows0, sg0)
    pltpu.async_copy(g_hbm.at[si_v.at[0, 1]], rows1, sg1)

    def body(i, _):
        t0 = 2 * i
        pltpu.make_async_copy(g_hbm.at[si_v.at[0, 0]], rows0, sg0).wait()
        pltpu.async_copy(rows0, acc_sh.at[di_v.at[t0]], ss0, add=True)
        pltpu.make_async_copy(g_hbm.at[si_v.at[0, 1]], rows1, sg1).wait()
        pltpu.async_copy(rows1, acc_sh.at[di_v.at[t0 + 1]], ss1, add=True)

        @pl.when(i + 1 < NPAIR)
        def _prefetch():
            slot = lax.rem(i + 1, 2)
            pltpu.make_async_copy(
                src_hbm.at[wid, pl.ds(0, 2)], si_v.at[0], s_si).wait()
            pltpu.make_async_copy(rows0, acc_sh.at[di_v.at[t0]], ss0).wait()
            pltpu.async_copy(g_hbm.at[si_v.at[slot, 0]], rows0, sg0)
            pltpu.make_async_copy(rows1, acc_sh.at[di_v.at[t0 + 1]], ss1).wait()
            pltpu.async_copy(g_hbm.at[si_v.at[slot, 1]], rows1, sg1)

            @pl.when(i + 2 < NPAIR)
            def _fetch_next():
                pltpu.async_copy(
                    src_hbm.at[wid, pl.ds(2 * i + 4, 2)],
                    si_v.at[lax.rem(i, 2)], s_si)

        return 0

    lax.fori_loop(0, NPAIR, body, 0)
    pltpu.make_async_copy(rows0, acc_sh.at[di_v.at[0]], ss0).wait()
    pltpu.make_async_copy(rows1, acc_sh.at[di_v.at[1]], ss1).wait()
    plsc.subcore_barrier()
    pltpu.sync_copy(
        acc_sh.at[pl.ds(s * ROWS_PER_TILE, ROWS_PER_TILE)],
        out_hbm.at[c, pl.ds(s * ROWS_PER_TILE, ROWS_PER_TILE)],
    )


@functools.cache
def _agg_kernel():
    mesh = plsc.VectorSubcoreMesh(
        core_axis_name="c", subcore_axis_name="s",
        num_cores=NC, num_subcores=NS,
    )
    return pl.kernel(
        _agg_body,
        out_type=jax.ShapeDtypeStruct((NC, ACC_ROWS, D), jnp.float32),
        mesh=mesh,
        scratch_types=[
            pltpu.VMEM((2, 2, CHUNK), jnp.int32),
            pltpu.VMEM((NCHUNK, CHUNK), jnp.int32),
            pltpu.VMEM((CHUNK, D), jnp.float32),
            pltpu.VMEM((CHUNK, D), jnp.float32),
            pltpu.VMEM_SHARED((ACC_ROWS, D), jnp.float32),
            pltpu.SemaphoreType.DMA,
            pltpu.SemaphoreType.DMA,
            pltpu.SemaphoreType.DMA,
            pltpu.SemaphoreType.DMA,
            pltpu.SemaphoreType.DMA,
        ],
    )


def _tc1_body(x_ref, w1_ref, b1_ref, w2_ref, b2_ref, h_ref):
    h = _leaky(jnp.dot(x_ref[...], w1_ref[...],
                       preferred_element_type=jnp.float32) + b1_ref[...])
    h_ref[...] = _leaky(jnp.dot(h, w2_ref[...],
                                preferred_element_type=jnp.float32) + b2_ref[...])


def _tc2_body(degp_ref, h_ref, wc_ref, g_ref, dinv_ref):
    deg = degp_ref[0, :N, 0:1] + degp_ref[1, :N, 0:1] + 1.0
    dinv = lax.rsqrt(deg)
    dinv_ref[...] = dinv
    g_ref[...] = dinv * jnp.dot(h_ref[...], wc_ref[...],
                                preferred_element_type=jnp.float32)


def _tc3_body(s_ref, g_ref, dinv_ref, bc_ref, wc_ref, emb_ref, g2_ref):
    ssum = s_ref[0, :N, :] + s_ref[1, :N, :] + g_ref[...]
    dinv = dinv_ref[...]
    emb = _leaky(dinv * ssum + bc_ref[...])
    emb_ref[...] = emb
    g2_ref[...] = dinv * jnp.dot(emb, wc_ref[...],
                                 preferred_element_type=jnp.float32)


def _tc4_body(s_ref, g_ref, dinv_ref, bc_ref, wp_ref, bp_ref, emb_ref, o_ref):
    ssum = s_ref[0, :N, :] + s_ref[1, :N, :] + g_ref[...]
    emb = _leaky(dinv_ref[...] * ssum + bc_ref[...])
    emb_ref[...] = emb
    o_ref[...] = jnp.dot(emb, wp_ref[...],
                         preferred_element_type=jnp.float32) + bp_ref[...]


_f32 = jnp.float32


def _tc1(x, W1, b1, W2, b2):
    return pl.pallas_call(
        _tc1_body, out_shape=jax.ShapeDtypeStruct((N, D), _f32)
    )(x, W1, b1, W2, b2)


def _tc2(degp, h, Wc):
    return pl.pallas_call(
        _tc2_body,
        out_shape=(jax.ShapeDtypeStruct((N, D), _f32),
                   jax.ShapeDtypeStruct((N, 1), _f32)),
    )(degp, h, Wc)


def _tc3(S, g, dinv, bc, Wc):
    return pl.pallas_call(
        _tc3_body,
        out_shape=(jax.ShapeDtypeStruct((N, D), _f32),
                   jax.ShapeDtypeStruct((N, D), _f32)),
    )(S, g, dinv, bc, Wc)


def _tc4(S, g, dinv, bc, Wp, bp):
    return pl.pallas_call(
        _tc4_body,
        out_shape=(jax.ShapeDtypeStruct((N, D), _f32),
                   jax.ShapeDtypeStruct((N, 1), _f32)),
    )(S, g, dinv, bc, Wp, bp)


@jax.jit
def kernel(x, edge_index, W1, b1, W2, b2, Wc1, bc1, Wc2, bc2, Wp, bp):
    pad = E_PAD - E
    pad_idx = jnp.arange(pad, dtype=jnp.int32)
    src = jnp.concatenate([edge_index[0], pad_idx % N])
    dst = jnp.concatenate([edge_index[1], N + pad_idx % (ACC_ROWS - N)])
    src = src.reshape(NW, NCHUNK, CHUNK)
    dst = dst.reshape(NW, NCHUNK, CHUNK)
    ones128 = jnp.ones((CHUNK, D), _f32)
    zrows = jnp.zeros((ROWS_PER_TILE, D), _f32)

    h = _tc1(x, W1, b1.reshape(1, D), W2, b2.reshape(1, D))
    degp = _deg_kernel()(dst, ones128, zrows)
    g1, dinv = _tc2(degp, h, Wc1)
    S1 = _agg_kernel()(g1, src, dst, zrows)
    emb1, g2 = _tc3(S1, g1, dinv, bc1.reshape(1, D), Wc2)
    S2 = _agg_kernel()(g2, src, dst, zrows)
    emb2, o = _tc4(S2, g2, dinv, bc2.reshape(1, D), Wp, bp.reshape(1, 1))
    return (o.reshape(N), emb1, emb2)


# speedup vs baseline: 2.7085x; 1.1075x over previous
import functools

import jax
import jax.numpy as jnp
from jax import lax
from jax.experimental import pallas as pl
from jax.experimental.pallas import tpu as pltpu
from jax.experimental.pallas import tpu_sc as plsc

N = 10000
E = 320000
D = 128

NC = 2
NS = 16
NW = NC * NS
CHUNK = 128
NBUF = 3
NITER = 27
NCHUNK = NBUF * NITER
EPW = CHUNK * NCHUNK
E_PAD = EPW * NW
GROWS = N + CHUNK
ACC_ROWS = N
RPT = 632
RPT_TAIL = ACC_ROWS - 15 * RPT
DEG_ROWS = 10240
DROW = DEG_ROWS // D


def _leaky(h):
    return jnp.where(h >= 0, h, 0.01 * h)


def _mesh():
    return plsc.VectorSubcoreMesh(
        core_axis_name="c", subcore_axis_name="s",
        num_cores=NC, num_subcores=NS,
    )


_DEG_RPT = DEG_ROWS // NS


def _deg_body(dst_hbm, ones_hbm, zero_hbm, out_hbm, idx_v, ones_v, acc_sh, sem):
    c = lax.axis_index("c")
    s = lax.axis_index("s")
    wid = c * NS + s
    pltpu.sync_copy(zero_hbm.at[pl.ds(0, _DEG_RPT)],
                    acc_sh.at[pl.ds(s * _DEG_RPT, _DEG_RPT)])
    pltpu.sync_copy(ones_hbm, ones_v)
    pltpu.sync_copy(dst_hbm.at[wid], idx_v)
    plsc.subcore_barrier()

    def body(t, _):
        pltpu.async_copy(ones_v, acc_sh.at[idx_v.at[t]], sem, add=True)
        return 0

    lax.fori_loop(0, NCHUNK, body, 0)

    def drain(t, _):
        pltpu.make_async_copy(ones_v, acc_sh.at[idx_v.at[0]], sem).wait()
        return 0

    lax.fori_loop(0, NCHUNK, drain, 0)
    plsc.subcore_barrier()
    pltpu.sync_copy(
        acc_sh.at[pl.ds(s * _DEG_RPT, _DEG_RPT)],
        out_hbm.at[c, pl.ds(s * _DEG_RPT, _DEG_RPT)],
    )


@functools.cache
def _deg_kernel():
    return pl.kernel(
        _deg_body,
        out_type=jax.ShapeDtypeStruct((NC, DEG_ROWS, D), jnp.float32),
        mesh=_mesh(),
        scratch_types=[
            pltpu.VMEM((NCHUNK, CHUNK), jnp.int32),
            pltpu.VMEM((CHUNK, D), jnp.float32),
            pltpu.VMEM_SHARED((DEG_ROWS, D), jnp.float32),
            pltpu.SemaphoreType.DMA,
        ],
    )


def _agg_body(g_hbm, src_hbm, dst_hbm, zrows_hbm, out_hbm,
              si_v, di_v, rows0, rows1, rows2, acc_sh,
              sg0, sg1, sg2, ss0, ss1, ss2, s_ei, s_di):
    c = lax.axis_index("c")
    s = lax.axis_index("s")
    wid = c * NS + s
    rows = (rows0, rows1, rows2)
    sg = (sg0, sg1, sg2)
    ss = (ss0, ss1, ss2)

    @pl.when(s < NS - 1)
    def _zero():
        pltpu.sync_copy(zrows_hbm.at[pl.ds(0, RPT)],
                        acc_sh.at[pl.ds(s * RPT, RPT)])

    @pl.when(s == NS - 1)
    def _zero_tail():
        pltpu.sync_copy(zrows_hbm.at[pl.ds(0, RPT_TAIL)],
                        acc_sh.at[pl.ds(15 * RPT, RPT_TAIL)])

    plsc.subcore_barrier()

    pltpu.sync_copy(src_hbm.at[0, wid], si_v.at[0])
    pltpu.sync_copy(dst_hbm.at[0, wid], di_v)
    pltpu.async_copy(src_hbm.at[1, wid], si_v.at[1], s_ei)
    for b in range(NBUF):
        pltpu.async_copy(g_hbm.at[si_v.at[0, b]], rows[b], sg[b])

    def body(i, _):
        p = lax.rem(i, 2)
        p2 = lax.rem(i + 1, 2)

        @pl.when(i > 0)
        def _wait_dst():
            pltpu.make_async_copy(dst_hbm.at[0, wid], di_v, s_di).wait()

        for b in range(NBUF):
            pltpu.make_async_copy(g_hbm.at[si_v.at[p, b]], rows[b],
                                  sg[b]).wait()
            pltpu.async_copy(rows[b], acc_sh.at[di_v.at[b]], ss[b],
                             add=True)

        @pl.when(i + 1 < NITER)
        def _next():
            pltpu.make_async_copy(src_hbm.at[0, wid], si_v.at[0], s_ei).wait()
            for b in range(NBUF):
                pltpu.make_async_copy(rows[b], acc_sh.at[di_v.at[b]],
                                      ss[b]).wait()
                pltpu.async_copy(g_hbm.at[si_v.at[p2, b]], rows[b], sg[b])
            pltpu.async_copy(dst_hbm.at[i + 1, wid], di_v, s_di)

            @pl.when(i + 2 < NITER)
            def _fetch():
                pltpu.async_copy(src_hbm.at[i + 2, wid], si_v.at[p], s_ei)

        return 0

    lax.fori_loop(0, NITER, body, 0)
    for b in range(NBUF):
        pltpu.make_async_copy(rows[b], acc_sh.at[di_v.at[b]], ss[b]).wait()
    plsc.subcore_barrier()

    @pl.when(s < NS - 1)
    def _out():
        pltpu.sync_copy(acc_sh.at[pl.ds(s * RPT, RPT)],
                        out_hbm.at[c, pl.ds(s * RPT, RPT)])

    @pl.when(s == NS - 1)
    def _out_tail():
        pltpu.sync_copy(acc_sh.at[pl.ds(15 * RPT, RPT_TAIL)],
                        out_hbm.at[c, pl.ds(15 * RPT, RPT_TAIL)])


@functools.cache
def _agg_kernel():
    return pl.kernel(
        _agg_body,
        out_type=jax.ShapeDtypeStruct((NC, ACC_ROWS, D), jnp.float32),
        mesh=_mesh(),
        scratch_types=[
            pltpu.VMEM((2, NBUF, CHUNK), jnp.int32),
            pltpu.VMEM((NBUF, CHUNK), jnp.int32),
            pltpu.VMEM((CHUNK, D), jnp.float32),
            pltpu.VMEM((CHUNK, D), jnp.float32),
            pltpu.VMEM((CHUNK, D), jnp.float32),
            pltpu.VMEM_SHARED((ACC_ROWS, D), jnp.float32),
            pltpu.SemaphoreType.DMA,
            pltpu.SemaphoreType.DMA,
            pltpu.SemaphoreType.DMA,
            pltpu.SemaphoreType.DMA,
            pltpu.SemaphoreType.DMA,
            pltpu.SemaphoreType.DMA,
            pltpu.SemaphoreType.DMA,
            pltpu.SemaphoreType.DMA,
        ],
    )


def _tc1_body(x_ref, w1_ref, b1_ref, w2_ref, b2_ref, h_ref):
    h = _leaky(jnp.dot(x_ref[...], w1_ref[...],
                       preferred_element_type=jnp.float32) + b1_ref[...])
    h_ref[...] = _leaky(jnp.dot(h, w2_ref[...],
                                preferred_element_type=jnp.float32) + b2_ref[...])


def _tc2_body(d0_ref, d1_ref, h_ref, wc_ref, g_ref, dinv_ref):
    deg = d0_ref[...] + d1_ref[...] + 1.0
    dinv = lax.rsqrt(deg)
    dinv_ref[...] = dinv
    g_ref[:N, :] = dinv * jnp.dot(h_ref[...], wc_ref[...],
                                  preferred_element_type=jnp.float32)
    g_ref[N:, :] = jnp.zeros((GROWS - N, D), jnp.float32)


def _tc3_body(s_ref, g_ref, dinv_ref, bc_ref, wc_ref, emb_ref, g2_ref):
    ssum = s_ref[0] + s_ref[1] + g_ref[:N, :]
    dinv = dinv_ref[...]
    emb = _leaky(dinv * ssum + bc_ref[...])
    emb_ref[...] = emb
    g2_ref[:N, :] = dinv * jnp.dot(emb, wc_ref[...],
                                   preferred_element_type=jnp.float32)
    g2_ref[N:, :] = jnp.zeros((GROWS - N, D), jnp.float32)


def _tc4_body(s_ref, g_ref, dinv_ref, bc_ref, wp_ref, bp_ref, emb_ref, o_ref):
    ssum = s_ref[0] + s_ref[1] + g_ref[:N, :]
    emb = _leaky(dinv_ref[...] * ssum + bc_ref[...])
    emb_ref[...] = emb
    o_ref[...] = jnp.dot(emb, wp_ref[...],
                         preferred_element_type=jnp.float32) + bp_ref[...]


_f32 = jnp.float32


def _tc1(x, W1, b1, W2, b2):
    return pl.pallas_call(
        _tc1_body, out_shape=jax.ShapeDtypeStruct((N, D), _f32)
    )(x, W1, b1, W2, b2)


def _tc2(d0, d1, h, Wc):
    return pl.pallas_call(
        _tc2_body,
        out_shape=(jax.ShapeDtypeStruct((GROWS, D), _f32),
                   jax.ShapeDtypeStruct((N, 1), _f32)),
    )(d0, d1, h, Wc)


def _tc3(S, g, dinv, bc, Wc):
    return pl.pallas_call(
        _tc3_body,
        out_shape=(jax.ShapeDtypeStruct((N, D), _f32),
                   jax.ShapeDtypeStruct((GROWS, D), _f32)),
    )(S, g, dinv, bc, Wc)


def _tc4(S, g, dinv, bc, Wp, bp):
    return pl.pallas_call(
        _tc4_body,
        out_shape=(jax.ShapeDtypeStruct((N, D), _f32),
                   jax.ShapeDtypeStruct((N, 1), _f32)),
    )(S, g, dinv, bc, Wp, bp)


@jax.jit
def kernel(x, edge_index, W1, b1, W2, b2, Wc1, bc1, Wc2, bc2, Wp, bp):
    pad = E_PAD - E
    pad_idx = jnp.arange(pad, dtype=jnp.int32)
    src = jnp.concatenate([edge_index[0], N + pad_idx % CHUNK])
    dst = jnp.concatenate([edge_index[1], pad_idx % N])
    src = src.reshape(NW, NITER, NBUF, CHUNK).transpose(1, 0, 2, 3)
    dst = dst.reshape(NW, NITER, NBUF, CHUNK).transpose(1, 0, 2, 3)
    dst_deg = jnp.concatenate([edge_index[1], N + pad_idx % (DEG_ROWS - N)])
    dst_deg = dst_deg.reshape(NW, NCHUNK, CHUNK)
    zrows = jnp.zeros((_DEG_RPT, D), _f32)
    ones128 = jnp.ones((CHUNK, D), _f32)

    h = _tc1(x, W1, b1.reshape(1, D), W2, b2.reshape(1, D))
    degp = _deg_kernel()(dst_deg, ones128, zrows)
    d0 = degp[0, :N, 0:1]
    d1 = degp[1, :N, 0:1]
    g1, dinv = _tc2(d0, d1, h, Wc1)
    S1 = _agg_kernel()(g1, src, dst, zrows)
    emb1, g2 = _tc3(S1, g1, dinv, bc1.reshape(1, D), Wc2)
    S2 = _agg_kernel()(g2, src, dst, zrows)
    emb2, o = _tc4(S2, g2, dinv, bc2.reshape(1, D), Wp, bp.reshape(1, 1))
    return (o.reshape(N), emb1, emb2)
